# baseline (speedup 1.0000x reference)
import jax
import jax.numpy as jnp
from jax.experimental import pallas as pl
from jax.experimental.pallas import tpu as pltpu

_K = 1024
_W = 64
_TB = 1024
_KC = 256


def _vq_block(x_ref, k_ref, xl_ref, xd_ref, stats_ref, iota_ref, iotab_ref,
              acc_ref):
    n = pl.program_id(0)
    t = pl.program_id(1)
    first = (n == 0) & (t == 0)
    last = (n == pl.num_programs(0) - 1) & (t == pl.num_programs(1) - 1)

    @pl.when(first)
    def _init():
        iota_ref[...] = jax.lax.broadcasted_iota(jnp.int32, (_KC, _TB), 0)
        iotab_ref[...] = jax.lax.broadcasted_iota(
            jnp.int32, (_KC, _TB), 0).astype(jnp.bfloat16)
        acc_ref[0] = 0.0
        acc_ref[1] = 0.0
        acc_ref[2] = 0.0

    xb = x_ref[0]
    k = k_ref[...]

    xbf = xb.astype(jnp.bfloat16)
    km2 = (k * -2.0).astype(jnp.bfloat16)
    x2 = jnp.sum(xb * xb, axis=0, keepdims=True)
    kk2 = jnp.sum(k * k, axis=1, keepdims=True)
    iota_l = iota_ref[...]

    mind = None
    midx = None
    for c in range(_K // _KC):
        lo, hi = c * _KC, (c + 1) * _KC
        kxc = jax.lax.dot_general(
            km2[lo:hi, :], xbf, (((1,), (0,)), ((), ())),
            preferred_element_type=jnp.float32)
        dc = (x2 + kxc) + kk2[lo:hi, :]
        mc = jnp.min(dc, axis=0, keepdims=True)
        ic = jnp.min(jnp.where(dc == mc, iota_l, _K), axis=0, keepdims=True)
        ic = ic + c * _KC
        if c == 0:
            mind, midx = mc, ic
        else:
            better = mc < mind
            midx = jnp.where(better, ic, midx)
            mind = jnp.minimum(mind, mc)

    iota_b = iotab_ref[...]
    xd = None
    for c in range(_K // _KC):
        lo, hi = c * _KC, (c + 1) * _KC
        onehot_c = jnp.where(
            iota_b == (midx - c * _KC).astype(jnp.bfloat16),
            jnp.bfloat16(1), jnp.bfloat16(0))
        xd_c = jax.lax.dot_general(
            k[lo:hi, :].astype(jnp.bfloat16), onehot_c,
            (((0,), (0,)), ((), ())),
            preferred_element_type=jnp.float32)
        xd = xd_c if xd is None else xd + xd_c

    xl_ref[0] = midx.astype(jnp.int32)
    xd_ref[0] = xd
    acc_ref[0] += jnp.sum(mind)
    acc_ref[1] += jnp.sum(xb)
    acc_ref[2] += jnp.sum(x2)

    @pl.when(last)
    def _fin():
        stats_ref[0] = acc_ref[0]
        stats_ref[1] = acc_ref[1]
        stats_ref[2] = acc_ref[2]


def kernel(x, k):
    N, W, T = x.shape
    gt = T // _TB
    grid = (N, gt)
    xl3, xd, stats = pl.pallas_call(
        _vq_block,
        grid=grid,
        in_specs=[
            pl.BlockSpec((1, W, _TB), lambda n, t: (n, 0, t)),
            pl.BlockSpec((_K, W), lambda n, t: (0, 0)),
        ],
        out_specs=[
            pl.BlockSpec((1, 1, _TB), lambda n, t: (n, 0, t)),
            pl.BlockSpec((1, W, _TB), lambda n, t: (n, 0, t)),
            pl.BlockSpec((3,), lambda n, t: (0,), memory_space=pltpu.SMEM),
        ],
        out_shape=[
            jax.ShapeDtypeStruct((N, 1, T), jnp.int32),
            jax.ShapeDtypeStruct((N, W, T), jnp.float32),
            jax.ShapeDtypeStruct((3,), jnp.float32),
        ],
        scratch_shapes=[
            pltpu.VMEM((_KC, _TB), jnp.int32),
            pltpu.VMEM((_KC, _TB), jnp.bfloat16),
            pltpu.SMEM((3,), jnp.float32),
        ],
        compiler_params=pltpu.CompilerParams(
            dimension_semantics=("arbitrary", "arbitrary")),
    )(x, k)

    numel = N * W * T
    ntok = N * T
    x_l = xl3.reshape(N, T)
    fit = stats[0] / ntok
    commit_loss = stats[0] / numel
    mean = stats[1] / numel
    prenorm = jnp.sqrt(jnp.maximum(stats[2] / numel - mean * mean, 0.0))
    return (x_l, xd, commit_loss, fit, prenorm)


# speedup vs baseline: 1.0739x; 1.0739x over previous
import jax
import jax.numpy as jnp
from jax.experimental import pallas as pl
from jax.experimental.pallas import tpu as pltpu

_K = 1024
_W = 64
_TB = 2048
_KC = 256


def _vq_block(x_ref, k_ref, xl_ref, xd_ref, stats_ref, iota_ref, acc_ref):
    n = pl.program_id(0)
    t = pl.program_id(1)
    first = (n == 0) & (t == 0)
    last = (n == pl.num_programs(0) - 1) & (t == pl.num_programs(1) - 1)

    @pl.when(first)
    def _init():
        iota_ref[...] = jax.lax.broadcasted_iota(jnp.int32, (_KC, _TB), 0)
        acc_ref[0] = 0.0
        acc_ref[1] = 0.0
        acc_ref[2] = 0.0

    xb = x_ref[0]
    k = k_ref[...]

    xbf = xb.astype(jnp.bfloat16)
    km2 = (k * -2.0).astype(jnp.bfloat16)
    x2 = jnp.sum(xb * xb, axis=0, keepdims=True)
    kk2 = jnp.sum(k * k, axis=1, keepdims=True)
    iota_l = iota_ref[...]

    mind = None
    midx = None
    for c in range(_K // _KC):
        lo, hi = c * _KC, (c + 1) * _KC
        kxc = jax.lax.dot_general(
            km2[lo:hi, :], xbf, (((1,), (0,)), ((), ())),
            preferred_element_type=jnp.float32)
        dc = (x2 + kxc) + kk2[lo:hi, :]
        mc = jnp.min(dc, axis=0, keepdims=True)
        ic = jnp.min(jnp.where(dc == mc, iota_l, _K), axis=0, keepdims=True)
        ic = ic + c * _KC
        if c == 0:
            mind, midx = mc, ic
        else:
            better = mc < mind
            midx = jnp.where(better, ic, midx)
            mind = jnp.minimum(mind, mc)

    xd = None
    for c in range(_K // _KC):
        lo, hi = c * _KC, (c + 1) * _KC
        onehot_c = (iota_l == (midx - c * _KC)).astype(jnp.bfloat16)
        xd_c = jax.lax.dot_general(
            k[lo:hi, :].astype(jnp.bfloat16), onehot_c,
            (((0,), (0,)), ((), ())),
            preferred_element_type=jnp.float32)
        xd = xd_c if xd is None else xd + xd_c

    xl_ref[0] = midx.astype(jnp.int32)
    xd_ref[0] = xd
    acc_ref[0] += jnp.sum(mind)
    acc_ref[1] += jnp.sum(xb)
    acc_ref[2] += jnp.sum(x2)

    @pl.when(last)
    def _fin():
        stats_ref[0] = acc_ref[0]
        stats_ref[1] = acc_ref[1]
        stats_ref[2] = acc_ref[2]


def kernel(x, k):
    N, W, T = x.shape
    gt = T // _TB
    grid = (N, gt)
    xl3, xd, stats = pl.pallas_call(
        _vq_block,
        grid=grid,
        in_specs=[
            pl.BlockSpec((1, W, _TB), lambda n, t: (n, 0, t)),
            pl.BlockSpec((_K, W), lambda n, t: (0, 0)),
        ],
        out_specs=[
            pl.BlockSpec((1, 1, _TB), lambda n, t: (n, 0, t)),
            pl.BlockSpec((1, W, _TB), lambda n, t: (n, 0, t)),
            pl.BlockSpec((3,), lambda n, t: (0,), memory_space=pltpu.SMEM),
        ],
        out_shape=[
            jax.ShapeDtypeStruct((N, 1, T), jnp.int32),
            jax.ShapeDtypeStruct((N, W, T), jnp.float32),
            jax.ShapeDtypeStruct((3,), jnp.float32),
        ],
        scratch_shapes=[
            pltpu.VMEM((_KC, _TB), jnp.int32),
            pltpu.SMEM((3,), jnp.float32),
        ],
        compiler_params=pltpu.CompilerParams(
            dimension_semantics=("arbitrary", "arbitrary")),
    )(x, k)

    numel = N * W * T
    ntok = N * T
    x_l = xl3.reshape(N, T)
    fit = stats[0] / ntok
    commit_loss = stats[0] / numel
    mean = stats[1] / numel
    prenorm = jnp.sqrt(jnp.maximum(stats[2] / numel - mean * mean, 0.0))
    return (x_l, xd, commit_loss, fit, prenorm)
